# baseline (speedup 1.0000x reference)
import jax
import jax.numpy as jnp
from jax import lax
from jax.experimental import pallas as pl
from jax.experimental.pallas import tpu as pltpu
from jax.experimental.pallas import tpu_sc as plsc

_NC = 2
_NS = 16
_NW = _NC * _NS
_L = 16
_T = 3000
_CPR = 16
_NB = 4
_NR = 8
_NSEG = _NB * _NR
_B = 64
_ROWS = _NB * _B * _NR
_SPW = _B // _NW
_W = 376
_CHUNKS = [(i * _W, _W) for i in range(7)] + [(7 * _W, _T - 7 * _W)]


def _sc_body(x_hbm, ri_hbm, out_hbm, ri_v, buf, outb, ss0, ss1, os0, os1):
    wid = lax.axis_index("s") * _NC + lax.axis_index("c")
    pltpu.sync_copy(ri_hbm, ri_v)
    ssem = (ss0, ss1)
    osem = (os0, os1)

    def stage(b, t, k):
        off, w = _CHUNKS[t]
        return pltpu.make_async_copy(
            x_hbm.at[b, :, pl.ds(off, w)], buf.at[k, :, pl.ds(0, w)], ssem[k]
        )

    def out_copies(b, t, k):
        off, w = _CHUNKS[t]
        return [
            pltpu.make_async_copy(
                outb.at[k, pl.ds(bin_ * _NR, _NR), pl.ds(0, w)],
                out_hbm.at[pl.ds(bin_ * (_B * _NR) + b * _NR, _NR),
                           pl.ds(off, w)],
                osem[k],
            )
            for bin_ in range(_NB)
        ]

    def reduce_chunk(t, k):
        _, w = _CHUNKS[t]

        def seg_body(seg, carry):
            row = ri_v[seg, :]
            cs = [row[j] for j in range(_CPR)]

            def col(o):
                vs = [buf[k, cs[j], pl.ds(o, _L)] for j in range(_CPR)]
                while len(vs) > 1:
                    vs = [vs[i] + vs[i + 1] for i in range(0, len(vs) - 1, 2)] \
                        + ([vs[-1]] if len(vs) % 2 else [])
                outb[k, seg, pl.ds(o, _L)] = vs[0] * (1.0 / _CPR)

            @plsc.parallel_loop(0, w // _L, unroll=4)
            def _col_loop(j):
                col(j * _L)
            if w % _L:
                col(w - _L)
            return carry

        lax.fori_loop(0, _NSEG, seg_body, 0)

    def process(b, t, k, nb, nt, prefetch, wait_out):
        stage(b, t, k).wait()
        if wait_out is not None:
            for c in out_copies(wait_out[0], wait_out[1], k):
                c.wait()
        reduce_chunk(t, k)
        if prefetch:
            stage(nb, nt, k).start()
        for c in out_copies(b, t, k):
            c.start()

    nchunk = len(_CHUNKS)
    for si in range(_SPW):
        b = wid * _SPW + si
        if si == 0:
            stage(b, 0, 0).start()
            stage(b, 1, 1).start()
        for t in range(nchunk):
            nb_, nt = (b, t + 2) if t + 2 < nchunk else (b + 1, t + 2 - nchunk)
            pf = t + 2 < nchunk or si + 1 < _SPW
            if t >= 2:
                wait_out = (b, t - 2)
            elif si > 0:
                wait_out = (b - 1, nchunk - 2 + t)
            else:
                wait_out = None
            process(b, t, t % 2, nb_, nt, pf, wait_out)

    bl = wid * _SPW + _SPW - 1
    for c in out_copies(bl, nchunk - 2, 0):
        c.wait()
    for c in out_copies(bl, nchunk - 1, 1):
        c.wait()


def kernel(x, region_indices):
    rif = region_indices.reshape(_NSEG, _CPR)
    mesh = plsc.VectorSubcoreMesh(core_axis_name="c", subcore_axis_name="s")
    out = pl.kernel(
        _sc_body,
        out_type=jax.ShapeDtypeStruct((_ROWS, _T), jnp.float32),
        mesh=mesh,
        scratch_types=[
            pltpu.VMEM((_NSEG, _CPR), jnp.int32),
            pltpu.VMEM((2, 128, _W), jnp.float32),
            pltpu.VMEM((2, _NSEG, _W), jnp.float32),
            pltpu.SemaphoreType.DMA,
            pltpu.SemaphoreType.DMA,
            pltpu.SemaphoreType.DMA,
            pltpu.SemaphoreType.DMA,
        ],
        compiler_params=pltpu.CompilerParams(use_tc_tiling_on_sc=False),
    )(x, rif)
    return out.reshape(_NB, _B, _NR, _T)


# speedup vs baseline: 1.0325x; 1.0325x over previous
import jax
import jax.numpy as jnp
from jax import lax
from jax.experimental import pallas as pl
from jax.experimental.pallas import tpu as pltpu
from jax.experimental.pallas import tpu_sc as plsc

_NC = 2
_NS = 16
_NW = _NC * _NS
_L = 16
_T = 3000
_CPR = 16
_NB = 4
_NR = 8
_NSEG = _NB * _NR
_B = 64
_ROWS = _NB * _B * _NR
_SPW = _B // _NW
_W = 376
_CHUNKS = [(i * _W, _W) for i in range(7)] + [(7 * _W, _T - 7 * _W)]


def _sc_body(x_hbm, ri_hbm, out_hbm, ri_v, buf, outb, ss0, ss1, os0, os1):
    wid = lax.axis_index("s") * _NC + lax.axis_index("c")
    pltpu.sync_copy(ri_hbm, ri_v)
    ssem = (ss0, ss1)
    osem = (os0, os1)

    def stage(b, t, k):
        off, w = _CHUNKS[t]
        return pltpu.make_async_copy(
            x_hbm.at[b, :, pl.ds(off, w)], buf.at[k, :, pl.ds(0, w)], ssem[k]
        )

    def out_copies(b, t, k):
        off, w = _CHUNKS[t]
        return [
            pltpu.make_async_copy(
                outb.at[k, pl.ds(bin_ * _NR, _NR), pl.ds(0, w)],
                out_hbm.at[pl.ds(bin_ * (_B * _NR) + b * _NR, _NR),
                           pl.ds(off, w)],
                osem[k],
            )
            for bin_ in range(_NB)
        ]

    def reduce_chunk(t, k):
        _, w = _CHUNKS[t]

        def seg_body(seg, carry):
            row = ri_v[seg, :]
            cs = [row[j] for j in range(_CPR)]

            def col(o):
                vs = [buf[k, cs[j], pl.ds(o, _L)] for j in range(_CPR)]
                while len(vs) > 1:
                    vs = [vs[i] + vs[i + 1] for i in range(0, len(vs) - 1, 2)] \
                        + ([vs[-1]] if len(vs) % 2 else [])
                outb[k, seg, pl.ds(o, _L)] = vs[0] * (1.0 / _CPR)

            @plsc.parallel_loop(0, w // _L, unroll=2)
            def _col_loop(j):
                col(j * _L)
            if w % _L:
                col(w - _L)
            return carry

        lax.fori_loop(0, _NSEG, seg_body, 0)

    def process(b, t, k, nb, nt, prefetch, wait_out):
        stage(b, t, k).wait()
        if wait_out is not None:
            for c in out_copies(wait_out[0], wait_out[1], k):
                c.wait()
        reduce_chunk(t, k)
        if prefetch:
            stage(nb, nt, k).start()
        for c in out_copies(b, t, k):
            c.start()

    nchunk = len(_CHUNKS)
    for si in range(_SPW):
        b = wid * _SPW + si
        if si == 0:
            stage(b, 0, 0).start()
            stage(b, 1, 1).start()
        for t in range(nchunk):
            nb_, nt = (b, t + 2) if t + 2 < nchunk else (b + 1, t + 2 - nchunk)
            pf = t + 2 < nchunk or si + 1 < _SPW
            if t >= 2:
                wait_out = (b, t - 2)
            elif si > 0:
                wait_out = (b - 1, nchunk - 2 + t)
            else:
                wait_out = None
            process(b, t, t % 2, nb_, nt, pf, wait_out)

    bl = wid * _SPW + _SPW - 1
    for c in out_copies(bl, nchunk - 2, 0):
        c.wait()
    for c in out_copies(bl, nchunk - 1, 1):
        c.wait()


def kernel(x, region_indices):
    rif = region_indices.reshape(_NSEG, _CPR)
    mesh = plsc.VectorSubcoreMesh(core_axis_name="c", subcore_axis_name="s")
    out = pl.kernel(
        _sc_body,
        out_type=jax.ShapeDtypeStruct((_ROWS, _T), jnp.float32),
        mesh=mesh,
        scratch_types=[
            pltpu.VMEM((_NSEG, _CPR), jnp.int32),
            pltpu.VMEM((2, 128, _W), jnp.float32),
            pltpu.VMEM((2, _NSEG, _W), jnp.float32),
            pltpu.SemaphoreType.DMA,
            pltpu.SemaphoreType.DMA,
            pltpu.SemaphoreType.DMA,
            pltpu.SemaphoreType.DMA,
        ],
        compiler_params=pltpu.CompilerParams(use_tc_tiling_on_sc=False),
    )(x, rif)
    return out.reshape(_NB, _B, _NR, _T)


# speedup vs baseline: 1.3925x; 1.3486x over previous
import jax
import jax.numpy as jnp
from jax import lax
from jax.experimental import pallas as pl
from jax.experimental.pallas import tpu as pltpu
from jax.experimental.pallas import tpu_sc as plsc

_NC = 2
_NS = 16
_NW = _NC * _NS
_L = 16
_T = 3000
_CPR = 16
_NB = 4
_NR = 8
_NSEG = _NB * _NR
_B = 64
_ROWS = _NB * _B * _NR
_SPW = _B // _NW
_W = 128
_NCH = _T // _W
_TAIL = _T - _NCH * _W
_NITEM = _SPW * _NCH


def _sc_body(x_hbm, ri_hbm, out_hbm, ri_v, buf, outb, ss0, ss1, os0, os1):
    wid = lax.axis_index("s") * _NC + lax.axis_index("c")
    pltpu.sync_copy(ri_hbm, ri_v)
    ssem = (ss0, ss1)
    osem = (os0, os1)

    def stage(i, k):
        b = wid * _SPW + i // _NCH
        off = (i % _NCH) * _W
        return pltpu.make_async_copy(
            x_hbm.at[b, :, pl.ds(off, _W)], buf.at[k], ssem[k]
        )

    def out_copies(i, k):
        b = wid * _SPW + i // _NCH
        off = (i % _NCH) * _W
        return [
            pltpu.make_async_copy(
                outb.at[k, pl.ds(bin_ * _NR, _NR), :],
                out_hbm.at[pl.ds(bin_ * (_B * _NR) + b * _NR, _NR),
                           pl.ds(off, _W)],
                osem[k],
            )
            for bin_ in range(_NB)
        ]

    def reduce_chunk(k):
        @plsc.parallel_loop(0, _NSEG)
        def _seg_loop(seg):
            row = ri_v[pl.ds(seg * _CPR, _CPR)]
            cs = [row[j] for j in range(_CPR)]
            for o in range(0, _W, _L):
                vs = [buf[k, cs[j], pl.ds(o, _L)] for j in range(_CPR)]
                while len(vs) > 1:
                    vs = [vs[i] + vs[i + 1] for i in range(0, len(vs) - 1, 2)] \
                        + ([vs[-1]] if len(vs) % 2 else [])
                outb[k, seg, pl.ds(o, _L)] = vs[0] * (1.0 / _CPR)

    def process(i, k, prefetch, wait_out):
        stage(i, k).wait()
        if wait_out:
            for c in out_copies(i, k):
                c.wait()
        reduce_chunk(k)
        if prefetch:
            stage(i + 2, k).start()
        for c in out_copies(i, k):
            c.start()

    stage(0, 0).start()
    stage(1, 1).start()
    process(0, 0, True, False)
    process(1, 1, True, False)

    def step(g, c):
        process(2 * g, 0, True, True)
        process(2 * g + 1, 1, True, True)
        return c

    lax.fori_loop(1, _NITEM // 2 - 1, step, 0)

    process(_NITEM - 2, 0, False, True)
    process(_NITEM - 1, 1, False, True)
    for c in out_copies(_NITEM - 2, 0):
        c.wait()
    for c in out_copies(_NITEM - 1, 1):
        c.wait()


def _tc_tail_body(xt_ref, ri_ref, o_ref):
    ri = ri_ref[...]
    cols = lax.broadcasted_iota(jnp.int32, (_NSEG, _CPR, 128), 2)
    one_hot = (ri[:, :, None] == cols).astype(jnp.float32)
    m = jnp.sum(one_hot, axis=1) * (1.0 / _CPR)
    xb = xt_ref[0]
    o_ref[0] = jnp.dot(m, xb, preferred_element_type=jnp.float32)


def _tc_tail(x, rif):
    xt = lax.slice(x, (0, 0, _NCH * _W), (_B, 128, _T))
    return pl.pallas_call(
        _tc_tail_body,
        grid=(_B,),
        in_specs=[
            pl.BlockSpec((1, 128, _TAIL), lambda b: (b, 0, 0)),
            pl.BlockSpec((_NSEG, _CPR), lambda b: (0, 0)),
        ],
        out_specs=pl.BlockSpec((1, _NSEG, _TAIL), lambda b: (b, 0, 0)),
        out_shape=jax.ShapeDtypeStruct((_B, _NSEG, _TAIL), jnp.float32),
    )(xt, rif)


def kernel(x, region_indices):
    rif = region_indices.reshape(_NSEG, _CPR)
    ri1 = region_indices.reshape(_NSEG * _CPR)
    mesh = plsc.VectorSubcoreMesh(core_axis_name="c", subcore_axis_name="s")
    main = pl.kernel(
        _sc_body,
        out_type=jax.ShapeDtypeStruct((_ROWS, _T), jnp.float32),
        mesh=mesh,
        scratch_types=[
            pltpu.VMEM((_NSEG * _CPR,), jnp.int32),
            pltpu.VMEM((2, 128, _W), jnp.float32),
            pltpu.VMEM((2, _NSEG, _W), jnp.float32),
            pltpu.SemaphoreType.DMA,
            pltpu.SemaphoreType.DMA,
            pltpu.SemaphoreType.DMA,
            pltpu.SemaphoreType.DMA,
        ],
        compiler_params=pltpu.CompilerParams(use_tc_tiling_on_sc=True),
    )(x, ri1)
    tail = _tc_tail(x, rif)
    main4 = main.reshape(_NB, _B, _NR, _T)
    tail4 = jnp.transpose(
        tail.reshape(_B, _NB, _NR, _TAIL), (1, 0, 2, 3)
    )
    return lax.dynamic_update_slice(main4, tail4, (0, 0, 0, _NCH * _W))
